# baseline (speedup 1.0000x reference)
import functools

import jax
import jax.numpy as jnp
from jax import lax
from jax.experimental import pallas as pl
from jax.experimental.pallas import tpu as pltpu
from jax.experimental.pallas import tpu_sc as plsc

N = 10000
D = 128
H = 64
E = 320000
ET = E + N

NC = 2
NS = 16
NW = NC * NS
BLK = 128
NBLK = 82
EPW = BLK * NBLK
EP = EPW * NW
NP = 10240
NPW = NP // NS
NBUF = 2
RB = 512


def _edge_body(h_hbm, asd_hbm, src_hbm, dst3_hbm,
               outp_hbm, denp_hbm,
               src_v, dst2_v, as_v, ad_v, den_v,
               gb0, gb1, sb0, sb1,
               out_sp,
               sg0, sg1, ss0, ss1):
    gbufs = (gb0, gb1)
    sbufs = (sb0, sb1)
    sems_g = (sg0, sg1)
    sems_s = (ss0, ss1)

    cidx = lax.axis_index("c")
    sidx = lax.axis_index("s")
    wid = sidx * NC + cidx
    base = wid * EPW

    pltpu.sync_copy(src_hbm.at[pl.ds(base, EPW)], src_v)
    pltpu.sync_copy(dst3_hbm.at[wid], dst2_v)
    pltpu.sync_copy(asd_hbm.at[0], as_v)
    pltpu.sync_copy(asd_hbm.at[1], ad_v)

    zeros16f = jnp.zeros((16,), jnp.float32)
    iota16 = lax.iota(jnp.int32, 16)
    zeros16i = jnp.zeros((16,), jnp.int32)

    def _zrow(i, carry):
        for j in range(4):
            gb0[i, pl.ds(j * 16, 16)] = zeros16f
        return carry
    lax.fori_loop(0, BLK, _zrow, 0)

    def _zden(i, carry):
        den_v[pl.ds(i * 16, 16)] = zeros16f
        return carry
    lax.fori_loop(0, NP // 16, _zden, 0)
    for k in range(NPW // BLK):
        r0 = sidx * NPW + k * BLK
        pltpu.sync_copy(gb0, out_sp.at[pl.ds(r0, BLK)])
    plsc.subcore_barrier()

    def _issue_gather(g, b):
        pltpu.async_copy(
            h_hbm.at[src_v.at[pl.ds(g * BLK, BLK)]], gbufs[b], sems_g[b])

    def _block_work(g, b, first, last):
        gb, sb = gbufs[b], sbufs[b]
        if not first:
            pltpu.make_async_copy(
                sb, out_sp.at[dst2_v.at[g - NBUF]], sems_s[b]).wait()
        pltpu.make_async_copy(
            h_hbm.at[src_v.at[pl.ds(g * BLK, BLK)]], gb, sems_g[b]).wait()

        def _grp(g8, carry):
            off = g * BLK + g8 * 16
            ids = src_v[pl.ds(off, 16)]
            idd = plsc.load_gather(
                dst2_v, [zeros16i + g, g8 * 16 + iota16])
            av = plsc.load_gather(as_v, [ids])
            dv = plsc.load_gather(ad_v, [idd])
            e = av + dv
            e = jnp.where(e >= 0.0, e, 0.2 * e)
            ee = jnp.exp(e)
            gid = base + off + iota16
            ee = jnp.where(gid < ET, ee, 0.0)
            plsc.addupdate_scatter(den_v, [idd], ee)
            for k in range(16):
                w = ee[k]
                r = g8 * 16 + k
                for j in range(4):
                    sb[r, pl.ds(j * 16, 16)] = gb[r, pl.ds(j * 16, 16)] * w
            return carry
        lax.fori_loop(0, 8, _grp, 0)

        if not last:
            _issue_gather(g + NBUF, b)
        pltpu.async_copy(sb, out_sp.at[dst2_v.at[g]], sems_s[b], add=True)

    for g in range(NBUF):
        _issue_gather(g, g)
    for g in range(NBUF):
        _block_work(g, g, first=True, last=False)

    def _steady(i, carry):
        g0 = NBUF + i * NBUF
        for b in range(NBUF):
            _block_work(g0 + b, b, first=False, last=False)
        return carry
    lax.fori_loop(0, (NBLK - 2 * NBUF) // NBUF, _steady, 0)

    for b in range(NBUF):
        _block_work(NBLK - NBUF + b, b, first=False, last=True)
    for b in range(NBUF):
        g = NBLK - NBUF + b
        pltpu.make_async_copy(
            sbufs[b], out_sp.at[dst2_v.at[g]], sems_s[b]).wait()
    pltpu.sync_copy(den_v, denp_hbm.at[wid])
    plsc.subcore_barrier()

    pltpu.sync_copy(out_sp.at[pl.ds(sidx * NPW, NPW)],
                    outp_hbm.at[cidx, sidx])


@functools.cache
def _make_edge_call():
    mesh = plsc.VectorSubcoreMesh(
        core_axis_name="c", subcore_axis_name="s",
        num_cores=NC, num_subcores=NS)
    return functools.partial(
        pl.kernel,
        out_type=(
            jax.ShapeDtypeStruct((NC, NS, NPW, H), jnp.float32),
            jax.ShapeDtypeStruct((NW, NP), jnp.float32),
        ),
        mesh=mesh,
        compiler_params=pltpu.CompilerParams(
            needs_layout_passes=False, use_tc_tiling_on_sc=False),
        scratch_types=(
            pltpu.VMEM((EPW,), jnp.int32),
            pltpu.VMEM((NBLK, BLK), jnp.int32),
            pltpu.VMEM((NP,), jnp.float32),
            pltpu.VMEM((NP,), jnp.float32),
            pltpu.VMEM((NP,), jnp.float32),
            pltpu.VMEM((BLK, H), jnp.float32),
            pltpu.VMEM((BLK, H), jnp.float32),
            pltpu.VMEM((BLK, H), jnp.float32),
            pltpu.VMEM((BLK, H), jnp.float32),
            pltpu.VMEM_SHARED((NP, H), jnp.float32),
            pltpu.SemaphoreType.DMA,
            pltpu.SemaphoreType.DMA,
            pltpu.SemaphoreType.DMA,
            pltpu.SemaphoreType.DMA,
        ),
    )(_edge_body)


def _prep_body(x_ref, w_ref, am_ref, h_ref, al_ref):
    h = jnp.dot(x_ref[...], w_ref[...], preferred_element_type=jnp.float32)
    h_ref[...] = h
    al_ref[...] = jnp.dot(h, am_ref[...], preferred_element_type=jnp.float32)


def _mid_body(op_ref, dp_ref, b_ref, w_ref, am_ref, h_ref, al_ref):
    raw = op_ref[0] + op_ref[1]
    den = jnp.sum(dp_ref[...], axis=0)
    act = raw / (den[:, None] + 1e-16) + b_ref[0:1, :]
    act = jnp.where(act >= 0.0, act, 0.01 * act)
    h = jnp.dot(act, w_ref[...], preferred_element_type=jnp.float32)
    h_ref[...] = h
    al_ref[...] = jnp.dot(h, am_ref[...], preferred_element_type=jnp.float32)


def _mlp_body(op_ref, dp_ref, b_ref, f1w_ref, f1b_ref, f2w_ref, f2b_ref,
              f3w_ref, f3b_ref, y_ref):
    raw = op_ref[0] + op_ref[1]
    den = jnp.sum(dp_ref[...], axis=0)
    act = raw / (den[:, None] + 1e-16) + b_ref[0:1, :]
    act = jnp.where(act >= 0.0, act, 0.01 * act)
    z = jnp.maximum(
        jnp.dot(act, f1w_ref[...], preferred_element_type=jnp.float32)
        + f1b_ref[0:1, :], 0.0)
    z = jnp.maximum(
        jnp.dot(z, f2w_ref[...], preferred_element_type=jnp.float32)
        + f2b_ref[0:1, :], 0.0)
    y_ref[...] = (jnp.dot(z, f3w_ref[...], preferred_element_type=jnp.float32)
                  + f3b_ref[0:1, :])


def _prep_call(x_pad, w, am):
    return pl.pallas_call(
        _prep_body,
        grid=(NP // RB,),
        in_specs=[
            pl.BlockSpec((RB, D), lambda i: (i, 0)),
            pl.BlockSpec((D, H), lambda i: (0, 0)),
            pl.BlockSpec((H, 8), lambda i: (0, 0)),
        ],
        out_specs=[
            pl.BlockSpec((RB, H), lambda i: (i, 0)),
            pl.BlockSpec((RB, 8), lambda i: (i, 0)),
        ],
        out_shape=[
            jax.ShapeDtypeStruct((NP, H), jnp.float32),
            jax.ShapeDtypeStruct((NP, 8), jnp.float32),
        ],
    )(x_pad, w, am)


def _mid_call(outp, denp, b_pad, w, am):
    return pl.pallas_call(
        _mid_body,
        grid=(NP // RB,),
        in_specs=[
            pl.BlockSpec((NC, RB, H), lambda i: (0, i, 0)),
            pl.BlockSpec((NW, RB), lambda i: (0, i)),
            pl.BlockSpec((8, H), lambda i: (0, 0)),
            pl.BlockSpec((H, H), lambda i: (0, 0)),
            pl.BlockSpec((H, 8), lambda i: (0, 0)),
        ],
        out_specs=[
            pl.BlockSpec((RB, H), lambda i: (i, 0)),
            pl.BlockSpec((RB, 8), lambda i: (i, 0)),
        ],
        out_shape=[
            jax.ShapeDtypeStruct((NP, H), jnp.float32),
            jax.ShapeDtypeStruct((NP, 8), jnp.float32),
        ],
    )(outp, denp, b_pad, w, am)


def _mlp_call(outp, denp, b_pad, f1w, f1b, f2w, f2b, f3w, f3b):
    return pl.pallas_call(
        _mlp_body,
        grid=(NP // RB,),
        in_specs=[
            pl.BlockSpec((NC, RB, H), lambda i: (0, i, 0)),
            pl.BlockSpec((NW, RB), lambda i: (0, i)),
            pl.BlockSpec((8, H), lambda i: (0, 0)),
            pl.BlockSpec((H, 100), lambda i: (0, 0)),
            pl.BlockSpec((8, 100), lambda i: (0, 0)),
            pl.BlockSpec((100, 50), lambda i: (0, 0)),
            pl.BlockSpec((8, 50), lambda i: (0, 0)),
            pl.BlockSpec((50, 128), lambda i: (0, 0)),
            pl.BlockSpec((8, 128), lambda i: (0, 0)),
        ],
        out_specs=pl.BlockSpec((RB, 128), lambda i: (i, 0)),
        out_shape=jax.ShapeDtypeStruct((NP, 128), jnp.float32),
    )(outp, denp, b_pad, f1w, f1b, f2w, f2b, f3w, f3b)


def _pad8(v, width):
    if v.shape[0] != width:
        v = jnp.pad(v, (0, width - v.shape[0]))
    return jnp.tile(v.reshape(1, -1), (8, 1))


def kernel(x, edge_index, W1, a_src1, a_dst1, b1, W2, a_src2, a_dst2, b2,
           fc1_w, fc1_b, fc2_w, fc2_b, fc3_w, fc3_b):
    loop = jnp.arange(N, dtype=jnp.int32)
    src = jnp.concatenate([edge_index[0].astype(jnp.int32), loop])
    dst = jnp.concatenate([edge_index[1].astype(jnp.int32), loop])
    src_pad = jnp.pad(src, (0, EP - ET))
    dst3 = jnp.pad(dst, (0, EP - ET)).reshape(NW, NBLK, BLK)

    x_pad = jnp.pad(x, ((0, NP - N), (0, 0)))
    am1 = jnp.zeros((H, 8), jnp.float32).at[:, 0].set(a_src1) \
        .at[:, 1].set(a_dst1)
    am2 = jnp.zeros((H, 8), jnp.float32).at[:, 0].set(a_src2) \
        .at[:, 1].set(a_dst2)

    h1, al1 = _prep_call(x_pad, W1, am1)
    h1, asd1, src_b, dst3_b = lax.optimization_barrier(
        (h1, al1.T, src_pad, dst3))
    outp1, denp1 = _make_edge_call()(h1, asd1, src_b, dst3_b)
    outp1, denp1 = lax.optimization_barrier(
        (outp1.reshape(NC, NP, H), denp1))
    h2, al2 = _mid_call(outp1, denp1, _pad8(b1, H), W2, am2)
    h2, asd2 = lax.optimization_barrier((h2, al2.T))
    outp2, denp2 = _make_edge_call()(h2, asd2, src_b, dst3_b)
    outp2, denp2 = lax.optimization_barrier(
        (outp2.reshape(NC, NP, H), denp2))
    y = _mlp_call(outp2, denp2, _pad8(b2, H),
                  fc1_w, _pad8(fc1_b, 100), fc2_w, _pad8(fc2_b, 50),
                  jnp.pad(fc3_w, ((0, 0), (0, 126))),
                  _pad8(fc3_b, 128))
    return y[:N, :2]


# speedup vs baseline: 1.2079x; 1.2079x over previous
import functools

import jax
import jax.numpy as jnp
from jax import lax
from jax.experimental import pallas as pl
from jax.experimental.pallas import tpu as pltpu
from jax.experimental.pallas import tpu_sc as plsc

N = 10000
D = 128
H = 64
E = 320000
ET = E + N

NC = 2
NS = 16
NW = NC * NS
BLK = 128
NBLK = 82
EPW = BLK * NBLK
EP = EPW * NW
NP = 10240
NPW = NP // NS
NBUF = 2
RB = 512


def _edge_body(h_hbm, asd_hbm, src_hbm, dst3_hbm,
               outp_hbm, denp_hbm,
               src_v, dst2_v, as_v, ad_v, den_v,
               gb0, gb1, sb0, sb1,
               out_sp,
               sg0, sg1, ss0, ss1):
    gbufs = (gb0, gb1)
    sbufs = (sb0, sb1)
    sems_g = (sg0, sg1)
    sems_s = (ss0, ss1)

    cidx = lax.axis_index("c")
    sidx = lax.axis_index("s")
    wid = sidx * NC + cidx
    base = wid * EPW

    pltpu.sync_copy(src_hbm.at[pl.ds(base, EPW)], src_v)
    pltpu.sync_copy(dst3_hbm.at[wid], dst2_v)
    pltpu.sync_copy(asd_hbm.at[0], as_v)
    pltpu.sync_copy(asd_hbm.at[1], ad_v)

    zeros16f = jnp.zeros((16,), jnp.float32)
    iota16 = lax.iota(jnp.int32, 16)
    zeros16i = jnp.zeros((16,), jnp.int32)

    def _zrow(i, carry):
        for j in range(4):
            gb0[i, pl.ds(j * 16, 16)] = zeros16f
        return carry
    lax.fori_loop(0, BLK, _zrow, 0)

    def _zden(i, carry):
        den_v[pl.ds(i * 16, 16)] = zeros16f
        return carry
    lax.fori_loop(0, NP // 16, _zden, 0)
    for k in range(NPW // BLK):
        r0 = sidx * NPW + k * BLK
        pltpu.sync_copy(gb0, out_sp.at[pl.ds(r0, BLK)])
    plsc.subcore_barrier()

    def _issue_gather(g, b):
        pltpu.async_copy(
            h_hbm.at[src_v.at[pl.ds(g * BLK, BLK)]], gbufs[b], sems_g[b])

    def _block_work(g, b):
        gb, sb = gbufs[b], sbufs[b]

        @pl.when(g >= NBUF)
        def _drain():
            pltpu.make_async_copy(
                sb, out_sp.at[dst2_v.at[g - NBUF]], sems_s[b]).wait()
        pltpu.make_async_copy(
            h_hbm.at[src_v.at[pl.ds(g * BLK, BLK)]], gb, sems_g[b]).wait()

        ees = []
        for g8 in range(8):
            off = g * BLK + g8 * 16
            ids = src_v[pl.ds(off, 16)]
            idd = plsc.load_gather(
                dst2_v, [zeros16i + g, g8 * 16 + iota16])
            av = plsc.load_gather(as_v, [ids])
            dv = plsc.load_gather(ad_v, [idd])
            e = av + dv
            e = jnp.where(e >= 0.0, e, 0.2 * e)
            ee = jnp.exp(e)
            gid = base + off + iota16
            ee = jnp.where(gid < ET, ee, 0.0)
            plsc.addupdate_scatter(den_v, [idd], ee)
            ees.append(ee)
        for g8 in range(8):
            for k in range(16):
                w = ees[g8][k]
                r = g8 * 16 + k
                for j in range(4):
                    sb[r, pl.ds(j * 16, 16)] = gb[r, pl.ds(j * 16, 16)] * w

        @pl.when(g < NBLK - NBUF)
        def _next():
            _issue_gather(g + NBUF, b)
        pltpu.async_copy(sb, out_sp.at[dst2_v.at[g]], sems_s[b], add=True)

    for g in range(NBUF):
        _issue_gather(g, g)

    def _steady(i, carry):
        for b in range(NBUF):
            _block_work(i * NBUF + b, b)
        return carry
    lax.fori_loop(0, NBLK // NBUF, _steady, 0)

    for b in range(NBUF):
        g = NBLK - NBUF + b
        pltpu.make_async_copy(
            sbufs[b], out_sp.at[dst2_v.at[g]], sems_s[b]).wait()
    pltpu.sync_copy(den_v, denp_hbm.at[wid])
    plsc.subcore_barrier()

    pltpu.sync_copy(out_sp.at[pl.ds(sidx * NPW, NPW)],
                    outp_hbm.at[cidx, sidx])


@functools.cache
def _make_edge_call():
    mesh = plsc.VectorSubcoreMesh(
        core_axis_name="c", subcore_axis_name="s",
        num_cores=NC, num_subcores=NS)
    return functools.partial(
        pl.kernel,
        out_type=(
            jax.ShapeDtypeStruct((NC, NS, NPW, H), jnp.float32),
            jax.ShapeDtypeStruct((NW, NP), jnp.float32),
        ),
        mesh=mesh,
        compiler_params=pltpu.CompilerParams(
            needs_layout_passes=False, use_tc_tiling_on_sc=False),
        scratch_types=(
            pltpu.VMEM((EPW,), jnp.int32),
            pltpu.VMEM((NBLK, BLK), jnp.int32),
            pltpu.VMEM((NP,), jnp.float32),
            pltpu.VMEM((NP,), jnp.float32),
            pltpu.VMEM((NP,), jnp.float32),
            pltpu.VMEM((BLK, H), jnp.float32),
            pltpu.VMEM((BLK, H), jnp.float32),
            pltpu.VMEM((BLK, H), jnp.float32),
            pltpu.VMEM((BLK, H), jnp.float32),
            pltpu.VMEM_SHARED((NP, H), jnp.float32),
            pltpu.SemaphoreType.DMA,
            pltpu.SemaphoreType.DMA,
            pltpu.SemaphoreType.DMA,
            pltpu.SemaphoreType.DMA,
        ),
    )(_edge_body)


def _prep_body(x_ref, w_ref, am_ref, h_ref, al_ref):
    h = jnp.dot(x_ref[...], w_ref[...], preferred_element_type=jnp.float32)
    h_ref[...] = h
    al_ref[...] = jnp.dot(h, am_ref[...], preferred_element_type=jnp.float32)


def _mid_body(op_ref, dp_ref, b_ref, w_ref, am_ref, h_ref, al_ref):
    raw = op_ref[0] + op_ref[1]
    den = jnp.sum(dp_ref[...], axis=0)
    act = raw / (den[:, None] + 1e-16) + b_ref[0:1, :]
    act = jnp.where(act >= 0.0, act, 0.01 * act)
    h = jnp.dot(act, w_ref[...], preferred_element_type=jnp.float32)
    h_ref[...] = h
    al_ref[...] = jnp.dot(h, am_ref[...], preferred_element_type=jnp.float32)


def _mlp_body(op_ref, dp_ref, b_ref, f1w_ref, f1b_ref, f2w_ref, f2b_ref,
              f3w_ref, f3b_ref, y_ref):
    raw = op_ref[0] + op_ref[1]
    den = jnp.sum(dp_ref[...], axis=0)
    act = raw / (den[:, None] + 1e-16) + b_ref[0:1, :]
    act = jnp.where(act >= 0.0, act, 0.01 * act)
    z = jnp.maximum(
        jnp.dot(act, f1w_ref[...], preferred_element_type=jnp.float32)
        + f1b_ref[0:1, :], 0.0)
    z = jnp.maximum(
        jnp.dot(z, f2w_ref[...], preferred_element_type=jnp.float32)
        + f2b_ref[0:1, :], 0.0)
    y_ref[...] = (jnp.dot(z, f3w_ref[...], preferred_element_type=jnp.float32)
                  + f3b_ref[0:1, :])


def _prep_call(x_pad, w, am):
    return pl.pallas_call(
        _prep_body,
        grid=(NP // RB,),
        in_specs=[
            pl.BlockSpec((RB, D), lambda i: (i, 0)),
            pl.BlockSpec((D, H), lambda i: (0, 0)),
            pl.BlockSpec((H, 8), lambda i: (0, 0)),
        ],
        out_specs=[
            pl.BlockSpec((RB, H), lambda i: (i, 0)),
            pl.BlockSpec((RB, 8), lambda i: (i, 0)),
        ],
        out_shape=[
            jax.ShapeDtypeStruct((NP, H), jnp.float32),
            jax.ShapeDtypeStruct((NP, 8), jnp.float32),
        ],
    )(x_pad, w, am)


def _mid_call(outp, denp, b_pad, w, am):
    return pl.pallas_call(
        _mid_body,
        grid=(NP // RB,),
        in_specs=[
            pl.BlockSpec((NC, RB, H), lambda i: (0, i, 0)),
            pl.BlockSpec((NW, RB), lambda i: (0, i)),
            pl.BlockSpec((8, H), lambda i: (0, 0)),
            pl.BlockSpec((H, H), lambda i: (0, 0)),
            pl.BlockSpec((H, 8), lambda i: (0, 0)),
        ],
        out_specs=[
            pl.BlockSpec((RB, H), lambda i: (i, 0)),
            pl.BlockSpec((RB, 8), lambda i: (i, 0)),
        ],
        out_shape=[
            jax.ShapeDtypeStruct((NP, H), jnp.float32),
            jax.ShapeDtypeStruct((NP, 8), jnp.float32),
        ],
    )(outp, denp, b_pad, w, am)


def _mlp_call(outp, denp, b_pad, f1w, f1b, f2w, f2b, f3w, f3b):
    return pl.pallas_call(
        _mlp_body,
        grid=(NP // RB,),
        in_specs=[
            pl.BlockSpec((NC, RB, H), lambda i: (0, i, 0)),
            pl.BlockSpec((NW, RB), lambda i: (0, i)),
            pl.BlockSpec((8, H), lambda i: (0, 0)),
            pl.BlockSpec((H, 100), lambda i: (0, 0)),
            pl.BlockSpec((8, 100), lambda i: (0, 0)),
            pl.BlockSpec((100, 50), lambda i: (0, 0)),
            pl.BlockSpec((8, 50), lambda i: (0, 0)),
            pl.BlockSpec((50, 128), lambda i: (0, 0)),
            pl.BlockSpec((8, 128), lambda i: (0, 0)),
        ],
        out_specs=pl.BlockSpec((RB, 128), lambda i: (i, 0)),
        out_shape=jax.ShapeDtypeStruct((NP, 128), jnp.float32),
    )(outp, denp, b_pad, f1w, f1b, f2w, f2b, f3w, f3b)


def _pad8(v, width):
    if v.shape[0] != width:
        v = jnp.pad(v, (0, width - v.shape[0]))
    return jnp.tile(v.reshape(1, -1), (8, 1))


def kernel(x, edge_index, W1, a_src1, a_dst1, b1, W2, a_src2, a_dst2, b2,
           fc1_w, fc1_b, fc2_w, fc2_b, fc3_w, fc3_b):
    loop = jnp.arange(N, dtype=jnp.int32)
    src = jnp.concatenate([edge_index[0].astype(jnp.int32), loop])
    dst = jnp.concatenate([edge_index[1].astype(jnp.int32), loop])
    src_pad = jnp.pad(src, (0, EP - ET))
    dst3 = jnp.pad(dst, (0, EP - ET)).reshape(NW, NBLK, BLK)

    x_pad = jnp.pad(x, ((0, NP - N), (0, 0)))
    am1 = jnp.zeros((H, 8), jnp.float32).at[:, 0].set(a_src1) \
        .at[:, 1].set(a_dst1)
    am2 = jnp.zeros((H, 8), jnp.float32).at[:, 0].set(a_src2) \
        .at[:, 1].set(a_dst2)

    h1, al1 = _prep_call(x_pad, W1, am1)
    h1, asd1, src_b, dst3_b = lax.optimization_barrier(
        (h1, al1.T, src_pad, dst3))
    outp1, denp1 = _make_edge_call()(h1, asd1, src_b, dst3_b)
    outp1, denp1 = lax.optimization_barrier(
        (outp1.reshape(NC, NP, H), denp1))
    h2, al2 = _mid_call(outp1, denp1, _pad8(b1, H), W2, am2)
    h2, asd2 = lax.optimization_barrier((h2, al2.T))
    outp2, denp2 = _make_edge_call()(h2, asd2, src_b, dst3_b)
    outp2, denp2 = lax.optimization_barrier(
        (outp2.reshape(NC, NP, H), denp2))
    y = _mlp_call(outp2, denp2, _pad8(b2, H),
                  fc1_w, _pad8(fc1_b, 100), fc2_w, _pad8(fc2_b, 50),
                  jnp.pad(fc3_w, ((0, 0), (0, 126))),
                  _pad8(fc3_b, 128))
    return y[:N, :2]


# speedup vs baseline: 1.4043x; 1.1626x over previous
import functools

import jax
import jax.numpy as jnp
from jax import lax
from jax.experimental import pallas as pl
from jax.experimental.pallas import tpu as pltpu
from jax.experimental.pallas import tpu_sc as plsc

N = 10000
D = 128
H = 64
E = 320000
ET = E + N

NC = 2
NS = 16
NW = NC * NS
BLK = 128
NBLK = 81
EPW = BLK * NBLK
EP = EPW * NW
NP = 10240
NPW = NP // NS
RB = 512


def _edge_body(h_hbm, asd_hbm, src_hbm, dst3_hbm,
               outp_hbm, denp_hbm,
               src_v, dst2_v, as_v, ad_v, gbuf, ee16,
               out_sp, den_sp, sem_g, sem_s, sem_d):
    cidx = lax.axis_index("c")
    sidx = lax.axis_index("s")
    wid = sidx * NC + cidx
    base = wid * EPW

    pltpu.sync_copy(src_hbm.at[pl.ds(base, EPW)], src_v)
    pltpu.sync_copy(dst3_hbm.at[wid], dst2_v)
    pltpu.sync_copy(asd_hbm.at[0], as_v)
    pltpu.sync_copy(asd_hbm.at[1], ad_v)

    zeros16f = jnp.zeros((16,), jnp.float32)

    def _zrow(i, carry):
        for j in range(4):
            gbuf[i, pl.ds(j * 16, 16)] = zeros16f
        ee16[i, pl.ds(0, 16)] = zeros16f
        return carry
    lax.fori_loop(0, BLK, _zrow, 0)
    for k in range(NPW // BLK):
        pltpu.sync_copy(gbuf, out_sp.at[pl.ds(sidx * NPW + k * BLK, BLK)])
        pltpu.sync_copy(ee16, den_sp.at[pl.ds(sidx * NPW + k * BLK, BLK)])
    plsc.subcore_barrier()

    iota16 = lax.iota(jnp.int32, 16)
    zeros16i = jnp.zeros((16,), jnp.int32)

    def _blk_body(blk, carry):
        gcp = pltpu.async_copy(
            h_hbm.at[src_v.at[pl.ds(blk * BLK, BLK)]], gbuf, sem_g)
        ees = []
        for g8 in range(8):
            off = blk * BLK + g8 * 16
            ids = src_v[pl.ds(off, 16)]
            idd = plsc.load_gather(dst2_v, [zeros16i + blk, g8 * 16 + iota16])
            av = plsc.load_gather(as_v, [ids])
            dv = plsc.load_gather(ad_v, [idd])
            e = av + dv
            e = jnp.where(e >= 0.0, e, 0.2 * e)
            ee = jnp.exp(e)
            gid = base + off + iota16
            ee = jnp.where(gid < ET, ee, 0.0)
            ees.append(ee)
            plsc.store_scatter(ee16, [g8 * 16 + iota16, zeros16i], ee)
        gcp.wait()
        for g8 in range(8):
            for k in range(16):
                w = ees[g8][k]
                r = g8 * 16 + k
                for j in range(4):
                    gbuf[r, pl.ds(j * 16, 16)] = (
                        gbuf[r, pl.ds(j * 16, 16)] * w)
        a1 = pltpu.async_copy(gbuf, out_sp.at[dst2_v.at[blk]], sem_s,
                              add=True)
        a2 = pltpu.async_copy(ee16, den_sp.at[dst2_v.at[blk]], sem_d,
                              add=True)
        a1.wait()
        a2.wait()
        return carry

    lax.fori_loop(0, NBLK, _blk_body, 0)
    plsc.subcore_barrier()

    for k in range(NPW // BLK):
        r0 = sidx * NPW + k * BLK
        pltpu.sync_copy(out_sp.at[pl.ds(r0, BLK)], gbuf)
        pltpu.sync_copy(gbuf, outp_hbm.at[cidx, pl.ds(r0, BLK)])
        pltpu.sync_copy(den_sp.at[pl.ds(r0, BLK)], ee16)
        pltpu.sync_copy(ee16, denp_hbm.at[cidx, pl.ds(r0, BLK)])


@functools.cache
def _make_edge_call():
    mesh = plsc.VectorSubcoreMesh(
        core_axis_name="c", subcore_axis_name="s",
        num_cores=NC, num_subcores=NS)
    return functools.partial(
        pl.kernel,
        out_type=(
            jax.ShapeDtypeStruct((NC, NP, H), jnp.float32),
            jax.ShapeDtypeStruct((NC, NP, 16), jnp.float32),
        ),
        mesh=mesh,
        compiler_params=pltpu.CompilerParams(
            needs_layout_passes=False, use_tc_tiling_on_sc=False),
        scratch_types=(
            pltpu.VMEM((EPW,), jnp.int32),
            pltpu.VMEM((NBLK, BLK), jnp.int32),
            pltpu.VMEM((NP,), jnp.float32),
            pltpu.VMEM((NP,), jnp.float32),
            pltpu.VMEM((BLK, H), jnp.float32),
            pltpu.VMEM((BLK, 16), jnp.float32),
            pltpu.VMEM_SHARED((NP, H), jnp.float32),
            pltpu.VMEM_SHARED((NP, 16), jnp.float32),
            pltpu.SemaphoreType.DMA,
            pltpu.SemaphoreType.DMA,
            pltpu.SemaphoreType.DMA,
        ),
    )(_edge_body)


def _prep_body(x_ref, w_ref, am_ref, h_ref, al_ref):
    h = jnp.dot(x_ref[...], w_ref[...], preferred_element_type=jnp.float32)
    h_ref[...] = h
    al_ref[...] = jnp.dot(h, am_ref[...], preferred_element_type=jnp.float32)


def _mid_body(op_ref, dp_ref, b_ref, w_ref, am_ref, h_ref, al_ref):
    raw = op_ref[0] + op_ref[1]
    den = dp_ref[0, :, 0] + dp_ref[1, :, 0]
    act = raw / (den[:, None] + 1e-16) + b_ref[0:1, :]
    act = jnp.where(act >= 0.0, act, 0.01 * act)
    h = jnp.dot(act, w_ref[...], preferred_element_type=jnp.float32)
    h_ref[...] = h
    al_ref[...] = jnp.dot(h, am_ref[...], preferred_element_type=jnp.float32)


def _mlp_body(op_ref, dp_ref, b_ref, f1w_ref, f1b_ref, f2w_ref, f2b_ref,
              f3w_ref, f3b_ref, y_ref):
    raw = op_ref[0] + op_ref[1]
    den = dp_ref[0, :, 0] + dp_ref[1, :, 0]
    act = raw / (den[:, None] + 1e-16) + b_ref[0:1, :]
    act = jnp.where(act >= 0.0, act, 0.01 * act)
    z = jnp.maximum(
        jnp.dot(act, f1w_ref[...], preferred_element_type=jnp.float32)
        + f1b_ref[0:1, :], 0.0)
    z = jnp.maximum(
        jnp.dot(z, f2w_ref[...], preferred_element_type=jnp.float32)
        + f2b_ref[0:1, :], 0.0)
    y_ref[...] = (jnp.dot(z, f3w_ref[...], preferred_element_type=jnp.float32)
                  + f3b_ref[0:1, :])


def _prep_call(x_pad, w, am):
    return pl.pallas_call(
        _prep_body,
        grid=(NP // RB,),
        in_specs=[
            pl.BlockSpec((RB, D), lambda i: (i, 0)),
            pl.BlockSpec((D, H), lambda i: (0, 0)),
            pl.BlockSpec((H, 8), lambda i: (0, 0)),
        ],
        out_specs=[
            pl.BlockSpec((RB, H), lambda i: (i, 0)),
            pl.BlockSpec((RB, 8), lambda i: (i, 0)),
        ],
        out_shape=[
            jax.ShapeDtypeStruct((NP, H), jnp.float32),
            jax.ShapeDtypeStruct((NP, 8), jnp.float32),
        ],
    )(x_pad, w, am)


def _mid_call(outp, denp, b_pad, w, am):
    return pl.pallas_call(
        _mid_body,
        grid=(NP // RB,),
        in_specs=[
            pl.BlockSpec((NC, RB, H), lambda i: (0, i, 0)),
            pl.BlockSpec((NC, RB, 16), lambda i: (0, i, 0)),
            pl.BlockSpec((8, H), lambda i: (0, 0)),
            pl.BlockSpec((H, H), lambda i: (0, 0)),
            pl.BlockSpec((H, 8), lambda i: (0, 0)),
        ],
        out_specs=[
            pl.BlockSpec((RB, H), lambda i: (i, 0)),
            pl.BlockSpec((RB, 8), lambda i: (i, 0)),
        ],
        out_shape=[
            jax.ShapeDtypeStruct((NP, H), jnp.float32),
            jax.ShapeDtypeStruct((NP, 8), jnp.float32),
        ],
    )(outp, denp, b_pad, w, am)


def _mlp_call(outp, denp, b_pad, f1w, f1b, f2w, f2b, f3w, f3b):
    return pl.pallas_call(
        _mlp_body,
        grid=(NP // RB,),
        in_specs=[
            pl.BlockSpec((NC, RB, H), lambda i: (0, i, 0)),
            pl.BlockSpec((NC, RB, 16), lambda i: (0, i, 0)),
            pl.BlockSpec((8, H), lambda i: (0, 0)),
            pl.BlockSpec((H, 100), lambda i: (0, 0)),
            pl.BlockSpec((8, 100), lambda i: (0, 0)),
            pl.BlockSpec((100, 50), lambda i: (0, 0)),
            pl.BlockSpec((8, 50), lambda i: (0, 0)),
            pl.BlockSpec((50, 128), lambda i: (0, 0)),
            pl.BlockSpec((8, 128), lambda i: (0, 0)),
        ],
        out_specs=pl.BlockSpec((RB, 128), lambda i: (i, 0)),
        out_shape=jax.ShapeDtypeStruct((NP, 128), jnp.float32),
    )(outp, denp, b_pad, f1w, f1b, f2w, f2b, f3w, f3b)


def _pad8(v, width):
    if v.shape[0] != width:
        v = jnp.pad(v, (0, width - v.shape[0]))
    return jnp.tile(v.reshape(1, -1), (8, 1))


def kernel(x, edge_index, W1, a_src1, a_dst1, b1, W2, a_src2, a_dst2, b2,
           fc1_w, fc1_b, fc2_w, fc2_b, fc3_w, fc3_b):
    loop = jnp.arange(N, dtype=jnp.int32)
    src = jnp.concatenate([edge_index[0].astype(jnp.int32), loop])
    dst = jnp.concatenate([edge_index[1].astype(jnp.int32), loop])
    src_pad = jnp.pad(src, (0, EP - ET))
    dst3 = jnp.pad(dst, (0, EP - ET)).reshape(NW, NBLK, BLK)

    x_pad = jnp.pad(x, ((0, NP - N), (0, 0)))
    am1 = jnp.zeros((H, 8), jnp.float32).at[:, 0].set(a_src1) \
        .at[:, 1].set(a_dst1)
    am2 = jnp.zeros((H, 8), jnp.float32).at[:, 0].set(a_src2) \
        .at[:, 1].set(a_dst2)

    h1, al1 = _prep_call(x_pad, W1, am1)
    outp1, denp1 = _make_edge_call()(h1, al1.T, src_pad, dst3)
    h2, al2 = _mid_call(outp1, denp1, _pad8(b1, H), W2, am2)
    outp2, denp2 = _make_edge_call()(h2, al2.T, src_pad, dst3)
    y = _mlp_call(outp2, denp2, _pad8(b2, H),
                  fc1_w, _pad8(fc1_b, 100), fc2_w, _pad8(fc2_b, 50),
                  jnp.pad(fc3_w, ((0, 0), (0, 126))),
                  _pad8(fc3_b, 128))
    return y[:N, :2]


# speedup vs baseline: 1.4774x; 1.0521x over previous
import functools

import jax
import jax.numpy as jnp
from jax import lax
from jax.experimental import pallas as pl
from jax.experimental.pallas import tpu as pltpu
from jax.experimental.pallas import tpu_sc as plsc

N = 10000
D = 128
H = 64
E = 320000
ET = E + N

NC = 2
NS = 16
NW = NC * NS
BLK = 128
NBLK = 81
EPW = BLK * NBLK
EP = EPW * NW
NP = 10240
NPW = NP // NS
RB = 512


def _edge_body(h_hbm, asd_hbm, src_hbm, dst3_hbm,
               outp_hbm, denp_hbm,
               src_v, dst2_v, as_v, ad_v, den_v, gbuf,
               out_sp, sem_g, sem_s):
    cidx = lax.axis_index("c")
    sidx = lax.axis_index("s")
    wid = sidx * NC + cidx
    base = wid * EPW

    pltpu.sync_copy(src_hbm.at[pl.ds(base, EPW)], src_v)
    pltpu.sync_copy(dst3_hbm.at[wid], dst2_v)
    pltpu.sync_copy(asd_hbm.at[0], as_v)
    pltpu.sync_copy(asd_hbm.at[1], ad_v)

    zeros16f = jnp.zeros((16,), jnp.float32)

    def _zrow(i, carry):
        for j in range(4):
            gbuf[i, pl.ds(j * 16, 16)] = zeros16f
        return carry
    lax.fori_loop(0, BLK, _zrow, 0)

    def _zden(i, carry):
        den_v[pl.ds(i * 16, 16)] = zeros16f
        return carry
    lax.fori_loop(0, NP // 16, _zden, 0)
    for k in range(NPW // BLK):
        pltpu.sync_copy(gbuf, out_sp.at[pl.ds(sidx * NPW + k * BLK, BLK)])
    plsc.subcore_barrier()

    iota16 = lax.iota(jnp.int32, 16)
    zeros16i = jnp.zeros((16,), jnp.int32)

    def _blk_body(blk, carry):
        gcp = pltpu.async_copy(
            h_hbm.at[src_v.at[pl.ds(blk * BLK, BLK)]], gbuf, sem_g)
        ees = []
        for g8 in range(8):
            off = blk * BLK + g8 * 16
            ids = src_v[pl.ds(off, 16)]
            idd = plsc.load_gather(dst2_v, [zeros16i + blk, g8 * 16 + iota16])
            av = plsc.load_gather(as_v, [ids])
            dv = plsc.load_gather(ad_v, [idd])
            e = av + dv
            e = jnp.where(e >= 0.0, e, 0.2 * e)
            ee = jnp.exp(e)
            gid = base + off + iota16
            ee = jnp.where(gid < ET, ee, 0.0)
            ees.append(ee)
            plsc.addupdate_scatter(den_v, [idd], ee)
        gcp.wait()
        for g8 in range(8):
            for k in range(16):
                w = ees[g8][k]
                r = g8 * 16 + k
                for j in range(4):
                    gbuf[r, pl.ds(j * 16, 16)] = (
                        gbuf[r, pl.ds(j * 16, 16)] * w)
        a1 = pltpu.async_copy(gbuf, out_sp.at[dst2_v.at[blk]], sem_s,
                              add=True)
        a1.wait()
        return carry

    lax.fori_loop(0, NBLK, _blk_body, 0)
    pltpu.sync_copy(den_v, denp_hbm.at[wid])
    plsc.subcore_barrier()

    for k in range(NPW // BLK):
        r0 = sidx * NPW + k * BLK
        pltpu.sync_copy(out_sp.at[pl.ds(r0, BLK)], gbuf)
        pltpu.sync_copy(gbuf, outp_hbm.at[cidx, pl.ds(r0, BLK)])


@functools.cache
def _make_edge_call():
    mesh = plsc.VectorSubcoreMesh(
        core_axis_name="c", subcore_axis_name="s",
        num_cores=NC, num_subcores=NS)
    return functools.partial(
        pl.kernel,
        out_type=(
            jax.ShapeDtypeStruct((NC, NP, H), jnp.float32),
            jax.ShapeDtypeStruct((NW, NP), jnp.float32),
        ),
        mesh=mesh,
        compiler_params=pltpu.CompilerParams(
            needs_layout_passes=False, use_tc_tiling_on_sc=False),
        scratch_types=(
            pltpu.VMEM((EPW,), jnp.int32),
            pltpu.VMEM((NBLK, BLK), jnp.int32),
            pltpu.VMEM((NP,), jnp.float32),
            pltpu.VMEM((NP,), jnp.float32),
            pltpu.VMEM((NP,), jnp.float32),
            pltpu.VMEM((BLK, H), jnp.float32),
            pltpu.VMEM_SHARED((NP, H), jnp.float32),
            pltpu.SemaphoreType.DMA,
            pltpu.SemaphoreType.DMA,
        ),
    )(_edge_body)


def _prep_body(x_ref, w_ref, am_ref, h_ref, al_ref):
    h = jnp.dot(x_ref[...], w_ref[...], preferred_element_type=jnp.float32)
    h_ref[...] = h
    al_ref[...] = jnp.dot(h, am_ref[...], preferred_element_type=jnp.float32)


def _mid_body(op_ref, dp_ref, b_ref, w_ref, am_ref, h_ref, al_ref):
    raw = op_ref[0] + op_ref[1]
    den = jnp.sum(dp_ref[...], axis=0)
    act = raw / (den[:, None] + 1e-16) + b_ref[0:1, :]
    act = jnp.where(act >= 0.0, act, 0.01 * act)
    h = jnp.dot(act, w_ref[...], preferred_element_type=jnp.float32)
    h_ref[...] = h
    al_ref[...] = jnp.dot(h, am_ref[...], preferred_element_type=jnp.float32)


def _mlp_body(op_ref, dp_ref, b_ref, f1w_ref, f1b_ref, f2w_ref, f2b_ref,
              f3w_ref, f3b_ref, y_ref):
    raw = op_ref[0] + op_ref[1]
    den = jnp.sum(dp_ref[...], axis=0)
    act = raw / (den[:, None] + 1e-16) + b_ref[0:1, :]
    act = jnp.where(act >= 0.0, act, 0.01 * act)
    z = jnp.maximum(
        jnp.dot(act, f1w_ref[...], preferred_element_type=jnp.float32)
        + f1b_ref[0:1, :], 0.0)
    z = jnp.maximum(
        jnp.dot(z, f2w_ref[...], preferred_element_type=jnp.float32)
        + f2b_ref[0:1, :], 0.0)
    y_ref[...] = (jnp.dot(z, f3w_ref[...], preferred_element_type=jnp.float32)
                  + f3b_ref[0:1, :])


def _prep_call(x_pad, w, am):
    return pl.pallas_call(
        _prep_body,
        grid=(NP // RB,),
        in_specs=[
            pl.BlockSpec((RB, D), lambda i: (i, 0)),
            pl.BlockSpec((D, H), lambda i: (0, 0)),
            pl.BlockSpec((H, 8), lambda i: (0, 0)),
        ],
        out_specs=[
            pl.BlockSpec((RB, H), lambda i: (i, 0)),
            pl.BlockSpec((RB, 8), lambda i: (i, 0)),
        ],
        out_shape=[
            jax.ShapeDtypeStruct((NP, H), jnp.float32),
            jax.ShapeDtypeStruct((NP, 8), jnp.float32),
        ],
    )(x_pad, w, am)


def _mid_call(outp, denp, b_pad, w, am):
    return pl.pallas_call(
        _mid_body,
        grid=(NP // RB,),
        in_specs=[
            pl.BlockSpec((NC, RB, H), lambda i: (0, i, 0)),
            pl.BlockSpec((NW, RB), lambda i: (0, i)),
            pl.BlockSpec((8, H), lambda i: (0, 0)),
            pl.BlockSpec((H, H), lambda i: (0, 0)),
            pl.BlockSpec((H, 8), lambda i: (0, 0)),
        ],
        out_specs=[
            pl.BlockSpec((RB, H), lambda i: (i, 0)),
            pl.BlockSpec((RB, 8), lambda i: (i, 0)),
        ],
        out_shape=[
            jax.ShapeDtypeStruct((NP, H), jnp.float32),
            jax.ShapeDtypeStruct((NP, 8), jnp.float32),
        ],
    )(outp, denp, b_pad, w, am)


def _mlp_call(outp, denp, b_pad, f1w, f1b, f2w, f2b, f3w, f3b):
    return pl.pallas_call(
        _mlp_body,
        grid=(NP // RB,),
        in_specs=[
            pl.BlockSpec((NC, RB, H), lambda i: (0, i, 0)),
            pl.BlockSpec((NW, RB), lambda i: (0, i)),
            pl.BlockSpec((8, H), lambda i: (0, 0)),
            pl.BlockSpec((H, 100), lambda i: (0, 0)),
            pl.BlockSpec((8, 100), lambda i: (0, 0)),
            pl.BlockSpec((100, 50), lambda i: (0, 0)),
            pl.BlockSpec((8, 50), lambda i: (0, 0)),
            pl.BlockSpec((50, 128), lambda i: (0, 0)),
            pl.BlockSpec((8, 128), lambda i: (0, 0)),
        ],
        out_specs=pl.BlockSpec((RB, 128), lambda i: (i, 0)),
        out_shape=jax.ShapeDtypeStruct((NP, 128), jnp.float32),
    )(outp, denp, b_pad, f1w, f1b, f2w, f2b, f3w, f3b)


def _pad8(v, width):
    if v.shape[0] != width:
        v = jnp.pad(v, (0, width - v.shape[0]))
    return jnp.tile(v.reshape(1, -1), (8, 1))


def kernel(x, edge_index, W1, a_src1, a_dst1, b1, W2, a_src2, a_dst2, b2,
           fc1_w, fc1_b, fc2_w, fc2_b, fc3_w, fc3_b):
    loop = jnp.arange(N, dtype=jnp.int32)
    src = jnp.concatenate([edge_index[0].astype(jnp.int32), loop])
    dst = jnp.concatenate([edge_index[1].astype(jnp.int32), loop])
    src_pad = jnp.pad(src, (0, EP - ET))
    dst3 = jnp.pad(dst, (0, EP - ET)).reshape(NW, NBLK, BLK)

    x_pad = jnp.pad(x, ((0, NP - N), (0, 0)))
    am1 = jnp.zeros((H, 8), jnp.float32).at[:, 0].set(a_src1) \
        .at[:, 1].set(a_dst1)
    am2 = jnp.zeros((H, 8), jnp.float32).at[:, 0].set(a_src2) \
        .at[:, 1].set(a_dst2)

    h1, al1 = _prep_call(x_pad, W1, am1)
    outp1, denp1 = _make_edge_call()(h1, al1.T, src_pad, dst3)
    h2, al2 = _mid_call(outp1, denp1, _pad8(b1, H), W2, am2)
    outp2, denp2 = _make_edge_call()(h2, al2.T, src_pad, dst3)
    y = _mlp_call(outp2, denp2, _pad8(b2, H),
                  fc1_w, _pad8(fc1_b, 100), fc2_w, _pad8(fc2_b, 50),
                  jnp.pad(fc3_w, ((0, 0), (0, 126))),
                  _pad8(fc3_b, 128))
    return y[:N, :2]
